# Initial kernel scaffold; baseline (speedup 1.0000x reference)
import jax
import jax.numpy as jnp
from jax.experimental import pallas as pl


def kernel(input_ids, embed_table, proj_w, proj_b):
    raise NotImplementedError("write your pallas kernel here")


# speedup vs baseline: 4.1460x; 4.1460x over previous
import jax
import jax.numpy as jnp
from jax.experimental import pallas as pl

_F32 = jnp.float32


def _body(ids_ref, emb_ref, pwt_ref, b_ref, hid_ref, log_ref):
    rblk = ids_ref.shape[0]
    ids = ids_ref[...].astype(_F32)

    j_row = jax.lax.broadcasted_iota(jnp.int32, (128, 16), 0)
    u_col = jax.lax.broadcasted_iota(jnp.int32, (128, 16), 1)
    p_sel = (j_row % 16 == u_col).astype(_F32)
    d_row = jax.lax.broadcasted_iota(jnp.int32, (16, 128), 0)
    i_col = jax.lax.broadcasted_iota(jnp.int32, (16, 128), 1)
    q_sel = (i_col % 16 == d_row).astype(_F32)
    jj = jax.lax.broadcasted_iota(jnp.int32, (128, 128), 0)
    ii = jax.lax.broadcasted_iota(jnp.int32, (128, 128), 1)
    blkmask = (jj // 16 == ii // 16).astype(_F32)
    k_r = jax.lax.broadcasted_iota(jnp.int32, (8, 128), 0)
    j_c = jax.lax.broadcasted_iota(jnp.int32, (8, 128), 1)
    expand = (j_c // 16 == k_r).astype(_F32)

    emb = emb_ref[...]
    ltab = jnp.dot(emb, pwt_ref[...], preferred_element_type=_F32) + b_ref[...]

    big_e = jnp.dot(jnp.dot(p_sel, emb, preferred_element_type=_F32), q_sel,
                    preferred_element_type=_F32) * blkmask
    big_l = jnp.dot(jnp.dot(p_sel, ltab, preferred_element_type=_F32), q_sel,
                    preferred_element_type=_F32) * blkmask

    idx_e = jnp.dot(ids, expand, preferred_element_type=_F32)
    vmod = (jax.lax.broadcasted_iota(jnp.int32, (rblk, 128), 1) % 16).astype(_F32)
    oh = (idx_e == vmod).astype(_F32)

    hid_ref[...] = jnp.dot(oh, big_e, preferred_element_type=_F32)
    log_ref[...] = jnp.dot(oh, big_l, preferred_element_type=_F32)


def kernel(input_ids, embed_table, proj_w, proj_b):
    bsz, seq = input_ids.shape
    d = embed_table.shape[1]
    tok = bsz * seq
    rows = tok // 8
    ids_w = input_ids.reshape(rows, 8).astype(jnp.int32)
    pwt = proj_w.T
    b_row = proj_b.reshape(1, d)

    rblk = 4096
    grid = rows // rblk

    hid_f, log_f = pl.pallas_call(
        _body,
        grid=(grid,),
        in_specs=[
            pl.BlockSpec((rblk, 8), lambda i: (i, 0)),
            pl.BlockSpec((16, 16), lambda i: (0, 0)),
            pl.BlockSpec((16, 16), lambda i: (0, 0)),
            pl.BlockSpec((1, 16), lambda i: (0, 0)),
        ],
        out_specs=[
            pl.BlockSpec((rblk, 128), lambda i: (i, 0)),
            pl.BlockSpec((rblk, 128), lambda i: (i, 0)),
        ],
        out_shape=[
            jax.ShapeDtypeStruct((rows, 128), _F32),
            jax.ShapeDtypeStruct((rows, 128), _F32),
        ],
    )(ids_w, embed_table, pwt, b_row)

    hidden = hid_f.reshape(bsz, seq, d)
    logits = log_f.reshape(bsz, seq, d)
    return (logits, hidden)


# speedup vs baseline: 98.0792x; 23.6563x over previous
import jax
import jax.numpy as jnp
from jax.experimental import pallas as pl

_F32 = jnp.float32


def _body(ids_ref, embt_ref, pw_ref, b_ref, hid_ref, log_ref):
    lblk = ids_ref.shape[0]
    nb = ids_ref.shape[1]

    embt = embt_ref[...]
    ltabt = jnp.dot(pw_ref[...], embt, preferred_element_type=_F32) + b_ref[...]
    tabt = jnp.concatenate([embt, ltabt], axis=0)

    iota_v = jax.lax.broadcasted_iota(jnp.int32, (16, nb), 0)
    for l in range(lblk):
        idrow = ids_ref[pl.ds(l, 1), :]
        oh = (jnp.broadcast_to(idrow, (16, nb)) == iota_v).astype(_F32)
        both = jnp.dot(tabt, oh, preferred_element_type=_F32)
        hid_ref[l, :, :] = both[0:16, :]
        log_ref[l, :, :] = both[16:32, :]


def kernel(input_ids, embed_table, proj_w, proj_b):
    bsz, seq = input_ids.shape
    d = embed_table.shape[1]
    ids_t = input_ids.T.astype(jnp.int32)
    embt = embed_table.T
    b_col = proj_b.reshape(d, 1)

    lblk = 8
    grid = seq // lblk

    hid_t, log_t = pl.pallas_call(
        _body,
        grid=(grid,),
        in_specs=[
            pl.BlockSpec((lblk, bsz), lambda i: (i, 0)),
            pl.BlockSpec((d, d), lambda i: (0, 0)),
            pl.BlockSpec((d, d), lambda i: (0, 0)),
            pl.BlockSpec((d, 1), lambda i: (0, 0)),
        ],
        out_specs=[
            pl.BlockSpec((lblk, d, bsz), lambda i: (i, 0, 0)),
            pl.BlockSpec((lblk, d, bsz), lambda i: (i, 0, 0)),
        ],
        out_shape=[
            jax.ShapeDtypeStruct((seq, d, bsz), _F32),
            jax.ShapeDtypeStruct((seq, d, bsz), _F32),
        ],
    )(ids_t, embt, proj_w, b_col)

    hidden = jnp.transpose(hid_t, (2, 0, 1))
    logits = jnp.transpose(log_t, (2, 0, 1))
    return (logits, hidden)
